# Initial kernel scaffold; baseline (speedup 1.0000x reference)
import jax
import jax.numpy as jnp
from jax.experimental import pallas as pl


def kernel(x, edge_index, W1, b1, W2, b2):
    raise NotImplementedError("write your pallas kernel here")


# speedup vs baseline: 3.4079x; 3.4079x over previous
import functools

import jax
import jax.numpy as jnp
from jax import lax
from jax.experimental import pallas as pl
from jax.experimental.pallas import tpu as pltpu
from jax.experimental.pallas import tpu_sc as plsc

MIN_NORM = 1e-7
EPS = 4e-3
N = 10000
E = 320000
D = 128
DH = D // 2

NC = 2
NS = 16
EPT = E // NS
CH = 80
STEPS = EPT // CH
RQ = 624
TAIL = N - NS * RQ

BR = 2000
GRID = N // BR

C_IN, C_HID, C_OUT = 1.0, 1.25, 1.5


def _norm(x):
    return jnp.sqrt(jnp.sum(x * x, axis=-1, keepdims=True))


def _artanh(x):
    x = jnp.clip(x, -1.0 + 1e-7, 1.0 - 1e-7)
    return 0.5 * jnp.log((1.0 + x) / (1.0 - x))


def _proj(x, c):
    norm = jnp.maximum(_norm(x), MIN_NORM)
    maxnorm = (1.0 - EPS) / jnp.sqrt(c)
    return jnp.where(norm > maxnorm, x / norm * maxnorm, x)


def _expmap0(u, c):
    sqrt_c = jnp.sqrt(c)
    u_norm = jnp.maximum(_norm(u), MIN_NORM)
    return jnp.tanh(sqrt_c * u_norm) * u / (sqrt_c * u_norm)


def _logmap0(p, c):
    sqrt_c = jnp.sqrt(c)
    p_norm = jnp.maximum(_norm(p), MIN_NORM)
    return _artanh(sqrt_c * p_norm) * p / (sqrt_c * p_norm)


def _mobius_add(x, y, c):
    x2 = jnp.sum(x * x, axis=-1, keepdims=True)
    y2 = jnp.sum(y * y, axis=-1, keepdims=True)
    xy = jnp.sum(x * y, axis=-1, keepdims=True)
    num = (1.0 + 2.0 * c * xy + c * y2) * x + (1.0 - c * x2) * y
    denom = 1.0 + 2.0 * c * xy + (c ** 2) * x2 * y2
    return num / jnp.maximum(denom, MIN_NORM)


def _hyp_linear(h, w_ref, b_ref, c):
    sqrt_c = jnp.sqrt(c)
    x_norm = jnp.maximum(_norm(h), MIN_NORM)
    mx = lax.dot_general(h, w_ref[...], (((1,), (1,)), ((), ())),
                         preferred_element_type=jnp.float32)
    mx_norm = jnp.maximum(_norm(mx), MIN_NORM)
    mv = jnp.tanh(mx_norm / x_norm * _artanh(sqrt_c * x_norm)) * mx / (mx_norm * sqrt_c)
    mv = _proj(mv, c)
    hyp_bias = _proj(_expmap0(b_ref[...], c), c)
    return _proj(_mobius_add(mv, hyp_bias, c), c)


def _k_pre(x_ref, w_ref, b_ref, oa_ref, ob_ref):
    h = _proj(_expmap0(x_ref[...], C_IN), C_IN)
    h = _hyp_linear(h, w_ref, b_ref, C_IN)
    xt = _logmap0(h, C_IN)
    oa_ref[...] = xt[:, :DH]
    ob_ref[...] = xt[:, DH:]


def _k_mid(pa_ref, pb_ref, deg_ref, w_ref, b_ref, oa_ref, ob_ref):
    agg = jnp.concatenate([pa_ref[...], pb_ref[...]], axis=-1)
    deg = deg_ref[:, 0:1]
    agg = agg / jnp.maximum(deg, 1.0)
    h = _proj(_expmap0(agg, C_IN), C_IN)
    xt = jax.nn.relu(_logmap0(h, C_IN))
    h = _proj(_expmap0(xt, C_HID), C_HID)
    h = _hyp_linear(h, w_ref, b_ref, C_HID)
    xt2 = _logmap0(h, C_HID)
    oa_ref[...] = xt2[:, :DH]
    ob_ref[...] = xt2[:, DH:]


def _k_post(pa_ref, pb_ref, deg_ref, o_ref):
    agg = jnp.concatenate([pa_ref[...], pb_ref[...]], axis=-1)
    deg = deg_ref[:, 0:1]
    agg = agg / jnp.maximum(deg, 1.0)
    h = _proj(_expmap0(agg, C_HID), C_HID)
    xt = jax.nn.relu(_logmap0(h, C_HID))
    o_ref[...] = _proj(_expmap0(xt, C_OUT), C_OUT)


def _row_spec():
    return pl.BlockSpec((BR, D), lambda i: (i, 0))


def _half_spec():
    return pl.BlockSpec((BR, DH), lambda i: (i, 0))


def _deg_spec():
    return pl.BlockSpec((BR, 16), lambda i: (i, 0))


def _w_spec():
    return pl.BlockSpec((D, D), lambda i: (0, 0))


def _b_spec():
    return pl.BlockSpec((1, D), lambda i: (0, 0))


_half_sds = jax.ShapeDtypeStruct((N, DH), jnp.float32)


def _tc_pre(x, w1, b1):
    return pl.pallas_call(
        _k_pre,
        grid=(GRID,),
        in_specs=[_row_spec(), _w_spec(), _b_spec()],
        out_specs=[_half_spec(), _half_spec()],
        out_shape=[_half_sds, _half_sds],
    )(x, w1, b1)


def _tc_mid(pa, pb, deg, w2, b2):
    return pl.pallas_call(
        _k_mid,
        grid=(GRID,),
        in_specs=[_half_spec(), _half_spec(), _deg_spec(), _w_spec(), _b_spec()],
        out_specs=[_half_spec(), _half_spec()],
        out_shape=[_half_sds, _half_sds],
    )(pa, pb, deg, w2, b2)


def _tc_post(pa, pb, deg):
    return pl.pallas_call(
        _k_post,
        grid=(GRID,),
        in_specs=[_half_spec(), _half_spec(), _deg_spec()],
        out_specs=_row_spec(),
        out_shape=jax.ShapeDtypeStruct((N, D), jnp.float32),
    )(pa, pb, deg)


@functools.cache
def _build_sc_aggregate():
    mesh = plsc.VectorSubcoreMesh(
        core_axis_name="c", subcore_axis_name="s",
        num_cores=NC, num_subcores=NS)
    return pl.kernel(
        _sc_aggregate_body,
        out_type=[
            jax.ShapeDtypeStruct((N, DH), jnp.float32),
            jax.ShapeDtypeStruct((N, DH), jnp.float32),
            jax.ShapeDtypeStruct((N, 16), jnp.float32),
        ],
        mesh=mesh,
        scratch_types=[
            pltpu.VMEM((CH,), jnp.int32),
            pltpu.VMEM((CH,), jnp.int32),
            pltpu.VMEM((CH, DH), jnp.float32),
            pltpu.VMEM((CH, 16), jnp.float32),
            pltpu.VMEM((RQ + TAIL, 16), jnp.float32),
            pltpu.VMEM_SHARED((N, DH), jnp.float32),
            pltpu.VMEM_SHARED((N, 16), jnp.float32),
            pltpu.SemaphoreType.DMA,
        ],
        compiler_params=pltpu.CompilerParams(use_tc_tiling_on_sc=False),
    )


def _sc_aggregate(xa, xb, src, dst):
    return _build_sc_aggregate()(xa, xb, src, dst)


def _sc_aggregate_body(xa_hbm, xb_hbm, src_hbm, dst_hbm,
                       agga_hbm, aggb_hbm, deg_hbm,
                       src_v, dst_v, rows_v, ones_v, zdeg_v,
                       agg_sh, deg_sh, sem):
    cid = lax.axis_index("c")
    sid = lax.axis_index("s")

    z16 = jnp.zeros((16,), jnp.float32)
    one16 = jnp.full((16,), 1.0, jnp.float32)

    def _fill_rows(i, _):
        r = i // (DH // 16)
        j = i % (DH // 16)
        rows_v[r, pl.ds(j * 16, 16)] = z16
        return 0
    lax.fori_loop(0, CH * (DH // 16), _fill_rows, 0)

    def _fill_ones(r, _):
        ones_v[r, :] = one16
        return 0
    lax.fori_loop(0, CH, _fill_ones, 0)

    def _fill_zdeg(r, _):
        zdeg_v[r, :] = z16
        return 0
    lax.fori_loop(0, RQ + TAIL, _fill_zdeg, 0)

    rbase = sid * RQ
    for k in range(RQ // CH):
        pltpu.sync_copy(rows_v, agg_sh.at[pl.ds(rbase + k * CH, CH)])
    rem = RQ % CH
    if rem:
        pltpu.sync_copy(rows_v.at[pl.ds(0, rem)],
                        agg_sh.at[pl.ds(rbase + (RQ // CH) * CH, rem)])

    @pl.when(sid == NS - 1)
    def _zero_agg_tail():
        pltpu.sync_copy(rows_v.at[pl.ds(0, TAIL)],
                        agg_sh.at[pl.ds(NS * RQ, TAIL)])

    @pl.when(cid == 0)
    def _zero_deg():
        pltpu.sync_copy(zdeg_v.at[pl.ds(0, RQ)], deg_sh.at[pl.ds(rbase, RQ)])

        @pl.when(sid == NS - 1)
        def _zero_deg_tail():
            pltpu.sync_copy(zdeg_v.at[pl.ds(0, TAIL)],
                            deg_sh.at[pl.ds(NS * RQ, TAIL)])

    plsc.subcore_barrier()

    base = sid * EPT

    def _run_edges(table_hbm, with_deg):
        def _step(st, _):
            off = base + st * CH
            pltpu.sync_copy(src_hbm.at[pl.ds(off, CH)], src_v)
            pltpu.sync_copy(dst_hbm.at[pl.ds(off, CH)], dst_v)
            pltpu.async_copy(table_hbm.at[src_v], rows_v, sem).wait()
            pltpu.sync_copy(rows_v, agg_sh.at[dst_v], add=True)
            if with_deg:
                pltpu.sync_copy(ones_v, deg_sh.at[dst_v], add=True)
            return 0
        lax.fori_loop(0, STEPS, _step, 0)

    @pl.when(cid == 0)
    def _edges_a():
        _run_edges(xa_hbm, True)

    @pl.when(cid == 1)
    def _edges_b():
        _run_edges(xb_hbm, False)

    plsc.subcore_barrier()

    def _read_out(out_hbm):
        pltpu.sync_copy(agg_sh.at[pl.ds(rbase, RQ)],
                        out_hbm.at[pl.ds(rbase, RQ)])

        @pl.when(sid == NS - 1)
        def _read_tail():
            pltpu.sync_copy(agg_sh.at[pl.ds(NS * RQ, TAIL)],
                            out_hbm.at[pl.ds(NS * RQ, TAIL)])

    @pl.when(cid == 0)
    def _read_a():
        _read_out(agga_hbm)
        pltpu.sync_copy(deg_sh.at[pl.ds(rbase, RQ)],
                        deg_hbm.at[pl.ds(rbase, RQ)])

        @pl.when(sid == NS - 1)
        def _read_deg_tail():
            pltpu.sync_copy(deg_sh.at[pl.ds(NS * RQ, TAIL)],
                            deg_hbm.at[pl.ds(NS * RQ, TAIL)])

    @pl.when(cid == 1)
    def _read_b():
        _read_out(aggb_hbm)


def kernel(x, edge_index, W1, b1, W2, b2):
    x = x.astype(jnp.float32)
    src = edge_index[0].astype(jnp.int32)
    dst = edge_index[1].astype(jnp.int32)
    b1r = b1.reshape(1, D).astype(jnp.float32)
    b2r = b2.reshape(1, D).astype(jnp.float32)

    xa1, xb1 = _tc_pre(x, W1, b1r)
    pa1, pb1, deg = _sc_aggregate(xa1, xb1, src, dst)
    xa2, xb2 = _tc_mid(pa1, pb1, deg, W2, b2r)
    pa2, pb2, _deg2 = _sc_aggregate(xa2, xb2, src, dst)
    return _tc_post(pa2, pb2, deg)


# speedup vs baseline: 6.7940x; 1.9936x over previous
import functools

import jax
import jax.numpy as jnp
from jax import lax
from jax.experimental import pallas as pl
from jax.experimental.pallas import tpu as pltpu
from jax.experimental.pallas import tpu_sc as plsc

MIN_NORM = 1e-7
EPS = 4e-3
N = 10000
E = 320000
D = 128
DH = D // 2

NC = 2
NS = 16
EPT = E // NS
CH = 80
STEPS = EPT // CH
RQ = 624
TAIL = N - NS * RQ

BR = 2000
GRID = N // BR

C_IN, C_HID, C_OUT = 1.0, 1.25, 1.5


def _norm(x):
    return jnp.sqrt(jnp.sum(x * x, axis=-1, keepdims=True))


def _artanh(x):
    x = jnp.clip(x, -1.0 + 1e-7, 1.0 - 1e-7)
    return 0.5 * jnp.log((1.0 + x) / (1.0 - x))


def _proj(x, c):
    norm = jnp.maximum(_norm(x), MIN_NORM)
    maxnorm = (1.0 - EPS) / jnp.sqrt(c)
    return jnp.where(norm > maxnorm, x / norm * maxnorm, x)


def _expmap0(u, c):
    sqrt_c = jnp.sqrt(c)
    u_norm = jnp.maximum(_norm(u), MIN_NORM)
    return jnp.tanh(sqrt_c * u_norm) * u / (sqrt_c * u_norm)


def _logmap0(p, c):
    sqrt_c = jnp.sqrt(c)
    p_norm = jnp.maximum(_norm(p), MIN_NORM)
    return _artanh(sqrt_c * p_norm) * p / (sqrt_c * p_norm)


def _mobius_add(x, y, c):
    x2 = jnp.sum(x * x, axis=-1, keepdims=True)
    y2 = jnp.sum(y * y, axis=-1, keepdims=True)
    xy = jnp.sum(x * y, axis=-1, keepdims=True)
    num = (1.0 + 2.0 * c * xy + c * y2) * x + (1.0 - c * x2) * y
    denom = 1.0 + 2.0 * c * xy + (c ** 2) * x2 * y2
    return num / jnp.maximum(denom, MIN_NORM)


def _hyp_linear(h, w_ref, b_ref, c):
    sqrt_c = jnp.sqrt(c)
    x_norm = jnp.maximum(_norm(h), MIN_NORM)
    mx = lax.dot_general(h, w_ref[...], (((1,), (1,)), ((), ())),
                         preferred_element_type=jnp.float32)
    mx_norm = jnp.maximum(_norm(mx), MIN_NORM)
    mv = jnp.tanh(mx_norm / x_norm * _artanh(sqrt_c * x_norm)) * mx / (mx_norm * sqrt_c)
    mv = _proj(mv, c)
    hyp_bias = _proj(_expmap0(b_ref[...], c), c)
    return _proj(_mobius_add(mv, hyp_bias, c), c)


def _k_pre(x_ref, w_ref, b_ref, oa_ref, ob_ref):
    h = _proj(_expmap0(x_ref[...], C_IN), C_IN)
    h = _hyp_linear(h, w_ref, b_ref, C_IN)
    xt = _logmap0(h, C_IN)
    oa_ref[...] = xt[:, :DH]
    ob_ref[...] = xt[:, DH:]


def _k_mid(pa_ref, pb_ref, deg_ref, w_ref, b_ref, oa_ref, ob_ref):
    agg = jnp.concatenate([pa_ref[...], pb_ref[...]], axis=-1)
    deg = deg_ref[:, 0:1]
    agg = agg / jnp.maximum(deg, 1.0)
    h = _proj(_expmap0(agg, C_IN), C_IN)
    xt = jax.nn.relu(_logmap0(h, C_IN))
    h = _proj(_expmap0(xt, C_HID), C_HID)
    h = _hyp_linear(h, w_ref, b_ref, C_HID)
    xt2 = _logmap0(h, C_HID)
    oa_ref[...] = xt2[:, :DH]
    ob_ref[...] = xt2[:, DH:]


def _k_post(pa_ref, pb_ref, deg_ref, o_ref):
    agg = jnp.concatenate([pa_ref[...], pb_ref[...]], axis=-1)
    deg = deg_ref[:, 0:1]
    agg = agg / jnp.maximum(deg, 1.0)
    h = _proj(_expmap0(agg, C_HID), C_HID)
    xt = jax.nn.relu(_logmap0(h, C_HID))
    o_ref[...] = _proj(_expmap0(xt, C_OUT), C_OUT)


def _row_spec():
    return pl.BlockSpec((BR, D), lambda i: (i, 0))


def _half_spec():
    return pl.BlockSpec((BR, DH), lambda i: (i, 0))


def _deg_spec():
    return pl.BlockSpec((BR, 16), lambda i: (i, 0))


def _w_spec():
    return pl.BlockSpec((D, D), lambda i: (0, 0))


def _b_spec():
    return pl.BlockSpec((1, D), lambda i: (0, 0))


_half_sds = jax.ShapeDtypeStruct((N, DH), jnp.float32)


def _tc_pre(x, w1, b1):
    return pl.pallas_call(
        _k_pre,
        grid=(GRID,),
        in_specs=[_row_spec(), _w_spec(), _b_spec()],
        out_specs=[_half_spec(), _half_spec()],
        out_shape=[_half_sds, _half_sds],
    )(x, w1, b1)


def _tc_mid(pa, pb, deg, w2, b2):
    return pl.pallas_call(
        _k_mid,
        grid=(GRID,),
        in_specs=[_half_spec(), _half_spec(), _deg_spec(), _w_spec(), _b_spec()],
        out_specs=[_half_spec(), _half_spec()],
        out_shape=[_half_sds, _half_sds],
    )(pa, pb, deg, w2, b2)


def _tc_post(pa, pb, deg):
    return pl.pallas_call(
        _k_post,
        grid=(GRID,),
        in_specs=[_half_spec(), _half_spec(), _deg_spec()],
        out_specs=_row_spec(),
        out_shape=jax.ShapeDtypeStruct((N, D), jnp.float32),
    )(pa, pb, deg)


@functools.cache
def _build_sc_aggregate():
    mesh = plsc.VectorSubcoreMesh(
        core_axis_name="c", subcore_axis_name="s",
        num_cores=NC, num_subcores=NS)
    return pl.kernel(
        _sc_aggregate_body,
        out_type=[
            jax.ShapeDtypeStruct((N, DH), jnp.float32),
            jax.ShapeDtypeStruct((N, DH), jnp.float32),
            jax.ShapeDtypeStruct((N, 16), jnp.float32),
        ],
        mesh=mesh,
        scratch_types=[
            pltpu.VMEM((STEPS, CH), jnp.int32),
            pltpu.VMEM((STEPS, CH), jnp.int32),
            pltpu.VMEM((CH, DH), jnp.float32),
            pltpu.VMEM((CH, DH), jnp.float32),
            pltpu.VMEM((CH, 16), jnp.float32),
            pltpu.VMEM((RQ + TAIL, 16), jnp.float32),
            pltpu.VMEM_SHARED((N, DH), jnp.float32),
            pltpu.VMEM_SHARED((N, 16), jnp.float32),
            pltpu.SemaphoreType.DMA,
            pltpu.SemaphoreType.DMA,
        ],
        compiler_params=pltpu.CompilerParams(use_tc_tiling_on_sc=False),
    )


def _sc_aggregate(xa, xb, src, dst):
    return _build_sc_aggregate()(xa, xb, src, dst)


def _sc_aggregate_body(xa_hbm, xb_hbm, src_hbm, dst_hbm,
                       agga_hbm, aggb_hbm, deg_hbm,
                       srcs_v, dsts_v, rows_a, rows_b, ones_v, zdeg_v,
                       agg_sh, deg_sh, sem_a, sem_b):
    cid = lax.axis_index("c")
    sid = lax.axis_index("s")

    z16 = jnp.zeros((16,), jnp.float32)
    one16 = jnp.full((16,), 1.0, jnp.float32)

    pltpu.sync_copy(src_hbm.at[sid], srcs_v)
    pltpu.sync_copy(dst_hbm.at[sid], dsts_v)

    def _fill_rows(i, _):
        r = i // (DH // 16)
        j = i % (DH // 16)
        rows_a[r, pl.ds(j * 16, 16)] = z16
        return 0
    lax.fori_loop(0, CH * (DH // 16), _fill_rows, 0)

    def _fill_ones(r, _):
        ones_v[r, :] = one16
        return 0
    lax.fori_loop(0, CH, _fill_ones, 0)

    def _fill_zdeg(r, _):
        zdeg_v[r, :] = z16
        return 0
    lax.fori_loop(0, RQ + TAIL, _fill_zdeg, 0)

    rbase = sid * RQ
    for k in range(RQ // CH):
        pltpu.sync_copy(rows_a, agg_sh.at[pl.ds(rbase + k * CH, CH)])
    rem = RQ % CH
    if rem:
        pltpu.sync_copy(rows_a.at[pl.ds(0, rem)],
                        agg_sh.at[pl.ds(rbase + (RQ // CH) * CH, rem)])

    @pl.when(sid == NS - 1)
    def _zero_agg_tail():
        pltpu.sync_copy(rows_a.at[pl.ds(0, TAIL)],
                        agg_sh.at[pl.ds(NS * RQ, TAIL)])

    @pl.when(cid == 0)
    def _zero_deg():
        pltpu.sync_copy(zdeg_v.at[pl.ds(0, RQ)], deg_sh.at[pl.ds(rbase, RQ)])

        @pl.when(sid == NS - 1)
        def _zero_deg_tail():
            pltpu.sync_copy(zdeg_v.at[pl.ds(0, TAIL)],
                            deg_sh.at[pl.ds(NS * RQ, TAIL)])

    plsc.subcore_barrier()

    def _run_edges(table_hbm, with_deg):
        def _issue(st, buf, sem):
            pltpu.async_copy(table_hbm.at[srcs_v.at[st]], buf, sem)

        def _wait(st, buf, sem):
            pltpu.make_async_copy(table_hbm.at[srcs_v.at[st]], buf, sem).wait()

        def _scatter(st, buf):
            pltpu.sync_copy(buf, agg_sh.at[dsts_v.at[st]], add=True)
            if with_deg:
                pltpu.sync_copy(ones_v, deg_sh.at[dsts_v.at[st]], add=True)

        _issue(0, rows_a, sem_a)

        def _pair(p, _):
            st_a = 2 * p
            _wait(st_a, rows_a, sem_a)
            _issue(st_a + 1, rows_b, sem_b)
            _scatter(st_a, rows_a)
            _wait(st_a + 1, rows_b, sem_b)

            @pl.when(p < STEPS // 2 - 1)
            def _next():
                _issue(st_a + 2, rows_a, sem_a)
            _scatter(st_a + 1, rows_b)
            return 0
        lax.fori_loop(0, STEPS // 2, _pair, 0)

    @pl.when(cid == 0)
    def _edges_a():
        _run_edges(xa_hbm, True)

    @pl.when(cid == 1)
    def _edges_b():
        _run_edges(xb_hbm, False)

    plsc.subcore_barrier()

    def _read_out(out_hbm):
        pltpu.sync_copy(agg_sh.at[pl.ds(rbase, RQ)],
                        out_hbm.at[pl.ds(rbase, RQ)])

        @pl.when(sid == NS - 1)
        def _read_tail():
            pltpu.sync_copy(agg_sh.at[pl.ds(NS * RQ, TAIL)],
                            out_hbm.at[pl.ds(NS * RQ, TAIL)])

    @pl.when(cid == 0)
    def _read_a():
        _read_out(agga_hbm)
        pltpu.sync_copy(deg_sh.at[pl.ds(rbase, RQ)],
                        deg_hbm.at[pl.ds(rbase, RQ)])

        @pl.when(sid == NS - 1)
        def _read_deg_tail():
            pltpu.sync_copy(deg_sh.at[pl.ds(NS * RQ, TAIL)],
                            deg_hbm.at[pl.ds(NS * RQ, TAIL)])

    @pl.when(cid == 1)
    def _read_b():
        _read_out(aggb_hbm)


def kernel(x, edge_index, W1, b1, W2, b2):
    x = x.astype(jnp.float32)
    src = edge_index[0].astype(jnp.int32).reshape(NS, STEPS, CH)
    dst = edge_index[1].astype(jnp.int32).reshape(NS, STEPS, CH)
    b1r = b1.reshape(1, D).astype(jnp.float32)
    b2r = b2.reshape(1, D).astype(jnp.float32)

    xa1, xb1 = _tc_pre(x, W1, b1r)
    pa1, pb1, deg = _sc_aggregate(xa1, xb1, src, dst)
    xa2, xb2 = _tc_mid(pa1, pb1, deg, W2, b2r)
    pa2, pb2, _deg2 = _sc_aggregate(xa2, xb2, src, dst)
    return _tc_post(pa2, pb2, deg)


# speedup vs baseline: 11.2401x; 1.6544x over previous
import functools

import jax
import jax.numpy as jnp
from jax import lax
from jax.experimental import pallas as pl
from jax.experimental.pallas import tpu as pltpu
from jax.experimental.pallas import tpu_sc as plsc

MIN_NORM = 1e-7
EPS = 4e-3
N = 10000
E = 320000
D = 128
DH = D // 2

NC = 2
NS = 16
EPT = E // NS
CH = 80
STEPS = EPT // CH
NBUF = 5
OUTER = STEPS // NBUF
HSTEP = STEPS // 2
RQ = 624
TAIL = N - NS * RQ

BR = 2000
GRID = N // BR

C_IN, C_HID, C_OUT = 1.0, 1.25, 1.5


def _norm(x):
    return jnp.sqrt(jnp.sum(x * x, axis=-1, keepdims=True))


def _artanh(x):
    x = jnp.clip(x, -1.0 + 1e-7, 1.0 - 1e-7)
    return 0.5 * jnp.log((1.0 + x) / (1.0 - x))


def _proj(x, c):
    norm = jnp.maximum(_norm(x), MIN_NORM)
    maxnorm = (1.0 - EPS) / jnp.sqrt(c)
    return jnp.where(norm > maxnorm, x / norm * maxnorm, x)


def _expmap0(u, c):
    sqrt_c = jnp.sqrt(c)
    u_norm = jnp.maximum(_norm(u), MIN_NORM)
    return jnp.tanh(sqrt_c * u_norm) * u / (sqrt_c * u_norm)


def _logmap0(p, c):
    sqrt_c = jnp.sqrt(c)
    p_norm = jnp.maximum(_norm(p), MIN_NORM)
    return _artanh(sqrt_c * p_norm) * p / (sqrt_c * p_norm)


def _mobius_add(x, y, c):
    x2 = jnp.sum(x * x, axis=-1, keepdims=True)
    y2 = jnp.sum(y * y, axis=-1, keepdims=True)
    xy = jnp.sum(x * y, axis=-1, keepdims=True)
    num = (1.0 + 2.0 * c * xy + c * y2) * x + (1.0 - c * x2) * y
    denom = 1.0 + 2.0 * c * xy + (c ** 2) * x2 * y2
    return num / jnp.maximum(denom, MIN_NORM)


def _hyp_linear(h, w_ref, b_ref, c):
    sqrt_c = jnp.sqrt(c)
    x_norm = jnp.maximum(_norm(h), MIN_NORM)
    mx = lax.dot_general(h, w_ref[...], (((1,), (1,)), ((), ())),
                         preferred_element_type=jnp.float32)
    mx_norm = jnp.maximum(_norm(mx), MIN_NORM)
    mv = jnp.tanh(mx_norm / x_norm * _artanh(sqrt_c * x_norm)) * mx / (mx_norm * sqrt_c)
    mv = _proj(mv, c)
    hyp_bias = _proj(_expmap0(b_ref[...], c), c)
    return _proj(_mobius_add(mv, hyp_bias, c), c)


def _k_pre(x_ref, w_ref, b_ref, oa_ref, ob_ref):
    h = _proj(_expmap0(x_ref[...], C_IN), C_IN)
    h = _hyp_linear(h, w_ref, b_ref, C_IN)
    xt = _logmap0(h, C_IN)
    oa_ref[...] = xt[:, :DH]
    ob_ref[...] = xt[:, DH:]


def _k_mid(pa_ref, pb_ref, deg_ref, w_ref, b_ref, oa_ref, ob_ref):
    agg = jnp.concatenate([pa_ref[...], pb_ref[...]], axis=-1)
    deg = deg_ref[0, :, 0:1] + deg_ref[1, :, 0:1]
    agg = agg / jnp.maximum(deg, 1.0)
    h = _proj(_expmap0(agg, C_IN), C_IN)
    xt = jax.nn.relu(_logmap0(h, C_IN))
    h = _proj(_expmap0(xt, C_HID), C_HID)
    h = _hyp_linear(h, w_ref, b_ref, C_HID)
    xt2 = _logmap0(h, C_HID)
    oa_ref[...] = xt2[:, :DH]
    ob_ref[...] = xt2[:, DH:]


def _k_post(pa_ref, pb_ref, deg_ref, o_ref):
    agg = jnp.concatenate([pa_ref[...], pb_ref[...]], axis=-1)
    deg = deg_ref[0, :, 0:1] + deg_ref[1, :, 0:1]
    agg = agg / jnp.maximum(deg, 1.0)
    h = _proj(_expmap0(agg, C_HID), C_HID)
    xt = jax.nn.relu(_logmap0(h, C_HID))
    o_ref[...] = _proj(_expmap0(xt, C_OUT), C_OUT)


def _row_spec():
    return pl.BlockSpec((BR, D), lambda i: (i, 0))


def _half_spec():
    return pl.BlockSpec((BR, DH), lambda i: (i, 0))


def _deg_spec():
    return pl.BlockSpec((NC, BR, 16), lambda i: (0, i, 0))


def _w_spec():
    return pl.BlockSpec((D, D), lambda i: (0, 0))


def _b_spec():
    return pl.BlockSpec((1, D), lambda i: (0, 0))


_half_sds = jax.ShapeDtypeStruct((N, DH), jnp.float32)


def _tc_pre(x, w1, b1):
    return pl.pallas_call(
        _k_pre,
        grid=(GRID,),
        in_specs=[_row_spec(), _w_spec(), _b_spec()],
        out_specs=[_half_spec(), _half_spec()],
        out_shape=[_half_sds, _half_sds],
    )(x, w1, b1)


def _tc_mid(pa, pb, deg, w2, b2):
    return pl.pallas_call(
        _k_mid,
        grid=(GRID,),
        in_specs=[_half_spec(), _half_spec(), _deg_spec(), _w_spec(), _b_spec()],
        out_specs=[_half_spec(), _half_spec()],
        out_shape=[_half_sds, _half_sds],
    )(pa, pb, deg, w2, b2)


def _tc_post(pa, pb, deg):
    return pl.pallas_call(
        _k_post,
        grid=(GRID,),
        in_specs=[_half_spec(), _half_spec(), _deg_spec()],
        out_specs=_row_spec(),
        out_shape=jax.ShapeDtypeStruct((N, D), jnp.float32),
    )(pa, pb, deg)


@functools.cache
def _build_sc_aggregate():
    mesh = plsc.VectorSubcoreMesh(
        core_axis_name="c", subcore_axis_name="s",
        num_cores=NC, num_subcores=NS)
    return pl.kernel(
        _sc_aggregate_body,
        out_type=[
            jax.ShapeDtypeStruct((N, DH), jnp.float32),
            jax.ShapeDtypeStruct((N, DH), jnp.float32),
            jax.ShapeDtypeStruct((NC, N, 16), jnp.float32),
        ],
        mesh=mesh,
        scratch_types=[
            pltpu.VMEM((STEPS, CH), jnp.int32),
            pltpu.VMEM((STEPS, CH), jnp.int32),
            [pltpu.VMEM((CH, DH), jnp.float32)] * NBUF,
            pltpu.VMEM((CH, 16), jnp.float32),
            pltpu.VMEM((RQ + TAIL, 16), jnp.float32),
            pltpu.VMEM_SHARED((N, DH), jnp.float32),
            pltpu.VMEM_SHARED((N, 16), jnp.float32),
            [pltpu.SemaphoreType.DMA] * NBUF,
        ],
        compiler_params=pltpu.CompilerParams(use_tc_tiling_on_sc=False),
    )


def _sc_aggregate(xa, xb, src, dst):
    return _build_sc_aggregate()(xa, xb, src, dst)


def _sc_aggregate_body(xa_hbm, xb_hbm, src_hbm, dst_hbm,
                       agga_hbm, aggb_hbm, deg_hbm,
                       srcs_v, dsts_v, rows, ones_v, zdeg_v,
                       agg_sh, deg_sh, sems):
    rows_a = rows[0]
    cid = lax.axis_index("c")
    sid = lax.axis_index("s")

    z16 = jnp.zeros((16,), jnp.float32)
    one16 = jnp.full((16,), 1.0, jnp.float32)

    pltpu.sync_copy(src_hbm.at[sid], srcs_v)
    pltpu.sync_copy(dst_hbm.at[sid], dsts_v)

    def _fill_rows(i, _):
        r = i // (DH // 16)
        j = i % (DH // 16)
        rows_a[r, pl.ds(j * 16, 16)] = z16
        return 0
    lax.fori_loop(0, CH * (DH // 16), _fill_rows, 0)

    def _fill_ones(r, _):
        ones_v[r, :] = one16
        return 0
    lax.fori_loop(0, CH, _fill_ones, 0)

    def _fill_zdeg(r, _):
        zdeg_v[r, :] = z16
        return 0
    lax.fori_loop(0, RQ + TAIL, _fill_zdeg, 0)

    rbase = sid * RQ
    for k in range(RQ // CH):
        pltpu.sync_copy(rows_a, agg_sh.at[pl.ds(rbase + k * CH, CH)])
    rem = RQ % CH
    if rem:
        pltpu.sync_copy(rows_a.at[pl.ds(0, rem)],
                        agg_sh.at[pl.ds(rbase + (RQ // CH) * CH, rem)])

    @pl.when(sid == NS - 1)
    def _zero_agg_tail():
        pltpu.sync_copy(rows_a.at[pl.ds(0, TAIL)],
                        agg_sh.at[pl.ds(NS * RQ, TAIL)])

    pltpu.sync_copy(zdeg_v.at[pl.ds(0, RQ)], deg_sh.at[pl.ds(rbase, RQ)])

    @pl.when(sid == NS - 1)
    def _zero_deg_tail():
        pltpu.sync_copy(zdeg_v.at[pl.ds(0, TAIL)],
                        deg_sh.at[pl.ds(NS * RQ, TAIL)])

    plsc.subcore_barrier()

    def _run_edges(table_hbm):
        def _issue(st, b):
            pltpu.async_copy(table_hbm.at[srcs_v.at[st]], rows[b], sems[b])

        def _wait(st, b):
            pltpu.make_async_copy(
                table_hbm.at[srcs_v.at[st]], rows[b], sems[b]).wait()

        def _scatter(st, b):
            pltpu.sync_copy(rows[b], agg_sh.at[dsts_v.at[st]], add=True)

            @pl.when(jnp.logical_xor(cid == 1, st < HSTEP))
            def _deg():
                pltpu.sync_copy(ones_v, deg_sh.at[dsts_v.at[st]], add=True)

        for b in range(NBUF):
            _issue(b, b)

        def _outer(q, _):
            for b in range(NBUF):
                st = q * NBUF + b
                _wait(st, b)
                _scatter(st, b)
                _issue(st + NBUF, b)
            return 0
        lax.fori_loop(0, OUTER - 1, _outer, 0)

        for b in range(NBUF):
            st = (OUTER - 1) * NBUF + b
            _wait(st, b)
            _scatter(st, b)

    @pl.when(cid == 0)
    def _edges_a():
        _run_edges(xa_hbm)

    @pl.when(cid == 1)
    def _edges_b():
        _run_edges(xb_hbm)

    plsc.subcore_barrier()

    def _read_out(out_hbm):
        pltpu.sync_copy(agg_sh.at[pl.ds(rbase, RQ)],
                        out_hbm.at[pl.ds(rbase, RQ)])

        @pl.when(sid == NS - 1)
        def _read_tail():
            pltpu.sync_copy(agg_sh.at[pl.ds(NS * RQ, TAIL)],
                            out_hbm.at[pl.ds(NS * RQ, TAIL)])

    @pl.when(cid == 0)
    def _read_a():
        _read_out(agga_hbm)

    @pl.when(cid == 1)
    def _read_b():
        _read_out(aggb_hbm)

    pltpu.sync_copy(deg_sh.at[pl.ds(rbase, RQ)],
                    deg_hbm.at[cid, pl.ds(rbase, RQ)])

    @pl.when(sid == NS - 1)
    def _read_deg_tail():
        pltpu.sync_copy(deg_sh.at[pl.ds(NS * RQ, TAIL)],
                        deg_hbm.at[cid, pl.ds(NS * RQ, TAIL)])


def kernel(x, edge_index, W1, b1, W2, b2):
    x = x.astype(jnp.float32)
    src = edge_index[0].astype(jnp.int32).reshape(NS, STEPS, CH)
    dst = edge_index[1].astype(jnp.int32).reshape(NS, STEPS, CH)
    b1r = b1.reshape(1, D).astype(jnp.float32)
    b2r = b2.reshape(1, D).astype(jnp.float32)

    xa1, xb1 = _tc_pre(x, W1, b1r)
    pa1, pb1, deg = _sc_aggregate(xa1, xb1, src, dst)
    xa2, xb2 = _tc_mid(pa1, pb1, deg, W2, b2r)
    pa2, pb2, _deg2 = _sc_aggregate(xa2, xb2, src, dst)
    return _tc_post(pa2, pb2, deg)


# speedup vs baseline: 12.3812x; 1.1015x over previous
import functools

import jax
import jax.numpy as jnp
from jax import lax
from jax.experimental import pallas as pl
from jax.experimental.pallas import tpu as pltpu
from jax.experimental.pallas import tpu_sc as plsc

MIN_NORM = 1e-7
EPS = 4e-3
N = 10000
E = 320000
D = 128
DH = D // 2

NC = 2
NS = 16
EPT = E // NS
CH = 80
STEPS = EPT // CH
NBUF = 5
OUTER = STEPS // NBUF
HSTEP = STEPS // 2
RQ = 624
TAIL = N - NS * RQ

BR = 2000
GRID = N // BR

C_IN, C_HID, C_OUT = 1.0, 1.25, 1.5


def _norm(x):
    return jnp.sqrt(jnp.sum(x * x, axis=-1, keepdims=True))


def _artanh(x):
    x = jnp.clip(x, -1.0 + 1e-7, 1.0 - 1e-7)
    return 0.5 * jnp.log((1.0 + x) / (1.0 - x))


def _proj(x, c):
    norm = jnp.maximum(_norm(x), MIN_NORM)
    maxnorm = (1.0 - EPS) / jnp.sqrt(c)
    return jnp.where(norm > maxnorm, x / norm * maxnorm, x)


def _expmap0(u, c):
    sqrt_c = jnp.sqrt(c)
    u_norm = jnp.maximum(_norm(u), MIN_NORM)
    return jnp.tanh(sqrt_c * u_norm) * u / (sqrt_c * u_norm)


def _logmap0(p, c):
    sqrt_c = jnp.sqrt(c)
    p_norm = jnp.maximum(_norm(p), MIN_NORM)
    return _artanh(sqrt_c * p_norm) * p / (sqrt_c * p_norm)


def _mobius_add(x, y, c):
    x2 = jnp.sum(x * x, axis=-1, keepdims=True)
    y2 = jnp.sum(y * y, axis=-1, keepdims=True)
    xy = jnp.sum(x * y, axis=-1, keepdims=True)
    num = (1.0 + 2.0 * c * xy + c * y2) * x + (1.0 - c * x2) * y
    denom = 1.0 + 2.0 * c * xy + (c ** 2) * x2 * y2
    return num / jnp.maximum(denom, MIN_NORM)


def _sumb(x, ones_ref):
    return lax.dot_general(x, ones_ref[...], (((1,), (0,)), ((), ())),
                           preferred_element_type=jnp.float32)


def _normb(x, ones_ref):
    return jnp.maximum(jnp.sqrt(_sumb(x * x, ones_ref)), MIN_NORM)


def _expmap0_projb(u, c, ones_ref):
    sqrt_c = jnp.sqrt(c)
    n = _normb(u, ones_ref)
    t = jnp.minimum(jnp.tanh(sqrt_c * n), 1.0 - EPS)
    return t * u / (sqrt_c * n)


def _logmap0b(p, c, ones_ref):
    sqrt_c = jnp.sqrt(c)
    n = _normb(p, ones_ref)
    return _artanh(sqrt_c * n) * p / (sqrt_c * n)


def _hyp_linear(h, w_ref, b_ref, c, ones_ref):
    sqrt_c = jnp.sqrt(c)
    x_norm = _normb(h, ones_ref)
    mx = lax.dot_general(h, w_ref[...], (((1,), (1,)), ((), ())),
                         preferred_element_type=jnp.float32)
    mx_norm = _normb(mx, ones_ref)
    t = jnp.minimum(jnp.tanh(mx_norm / x_norm * _artanh(sqrt_c * x_norm)),
                    1.0 - EPS)
    mv = t * mx / (mx_norm * sqrt_c)
    hyp_bias = _proj(_expmap0(b_ref[...], c), c)
    x2 = _sumb(mv * mv, ones_ref)
    y2 = jnp.sum(hyp_bias * hyp_bias, axis=-1, keepdims=True)
    xy = _sumb(mv * hyp_bias, ones_ref)
    num = (1.0 + 2.0 * c * xy + c * y2) * mv + (1.0 - c * x2) * hyp_bias
    denom = 1.0 + 2.0 * c * xy + (c ** 2) * x2 * y2
    ma = num / jnp.maximum(denom, MIN_NORM)
    n = _normb(ma, ones_ref)
    maxnorm = (1.0 - EPS) / sqrt_c
    return jnp.where(n > maxnorm, ma / n * maxnorm, ma)


def _k_pre(x_ref, w_ref, b_ref, ones_ref, oa_ref, ob_ref):
    h = _expmap0_projb(x_ref[...], C_IN, ones_ref)
    h = _hyp_linear(h, w_ref, b_ref, C_IN, ones_ref)
    xt = _logmap0b(h, C_IN, ones_ref)
    oa_ref[...] = xt[:, :DH]
    ob_ref[...] = xt[:, DH:]


def _k_mid(pa_ref, pb_ref, deg_ref, w_ref, b_ref, ones_ref, oa_ref, ob_ref):
    agg = jnp.concatenate([pa_ref[...], pb_ref[...]], axis=-1)
    deg = deg_ref[0, :, 0:1] + deg_ref[1, :, 0:1]
    agg = agg / jnp.maximum(deg, 1.0)
    h = _expmap0_projb(agg, C_IN, ones_ref)
    xt = jax.nn.relu(_logmap0b(h, C_IN, ones_ref))
    h = _expmap0_projb(xt, C_HID, ones_ref)
    h = _hyp_linear(h, w_ref, b_ref, C_HID, ones_ref)
    xt2 = _logmap0b(h, C_HID, ones_ref)
    oa_ref[...] = xt2[:, :DH]
    ob_ref[...] = xt2[:, DH:]


def _k_post(pa_ref, pb_ref, deg_ref, ones_ref, o_ref):
    agg = jnp.concatenate([pa_ref[...], pb_ref[...]], axis=-1)
    deg = deg_ref[0, :, 0:1] + deg_ref[1, :, 0:1]
    agg = agg / jnp.maximum(deg, 1.0)
    h = _expmap0_projb(agg, C_HID, ones_ref)
    xt = jax.nn.relu(_logmap0b(h, C_HID, ones_ref))
    o_ref[...] = _expmap0_projb(xt, C_OUT, ones_ref)


def _row_spec():
    return pl.BlockSpec((BR, D), lambda i: (i, 0))


def _half_spec():
    return pl.BlockSpec((BR, DH), lambda i: (i, 0))


def _deg_spec():
    return pl.BlockSpec((NC, BR, 16), lambda i: (0, i, 0))


def _w_spec():
    return pl.BlockSpec((D, D), lambda i: (0, 0))


def _b_spec():
    return pl.BlockSpec((1, D), lambda i: (0, 0))


_half_sds = jax.ShapeDtypeStruct((N, DH), jnp.float32)


def _ones_dd():
    return jnp.ones((D, D), jnp.float32)


def _tc_pre(x, w1, b1):
    return pl.pallas_call(
        _k_pre,
        grid=(GRID,),
        in_specs=[_row_spec(), _w_spec(), _b_spec(), _w_spec()],
        out_specs=[_half_spec(), _half_spec()],
        out_shape=[_half_sds, _half_sds],
    )(x, w1, b1, _ones_dd())


def _tc_mid(pa, pb, deg, w2, b2):
    return pl.pallas_call(
        _k_mid,
        grid=(GRID,),
        in_specs=[_half_spec(), _half_spec(), _deg_spec(), _w_spec(),
                  _b_spec(), _w_spec()],
        out_specs=[_half_spec(), _half_spec()],
        out_shape=[_half_sds, _half_sds],
    )(pa, pb, deg, w2, b2, _ones_dd())


def _tc_post(pa, pb, deg):
    return pl.pallas_call(
        _k_post,
        grid=(GRID,),
        in_specs=[_half_spec(), _half_spec(), _deg_spec(), _w_spec()],
        out_specs=_row_spec(),
        out_shape=jax.ShapeDtypeStruct((N, D), jnp.float32),
    )(pa, pb, deg, _ones_dd())


@functools.cache
def _build_sc_aggregate(want_deg):
    mesh = plsc.VectorSubcoreMesh(
        core_axis_name="c", subcore_axis_name="s",
        num_cores=NC, num_subcores=NS)
    out_type = [
        jax.ShapeDtypeStruct((N, DH), jnp.float32),
        jax.ShapeDtypeStruct((N, DH), jnp.float32),
    ]
    scratch_types = [
        pltpu.VMEM((STEPS, CH), jnp.int32),
        pltpu.VMEM((STEPS, CH), jnp.int32),
        [pltpu.VMEM((CH, DH), jnp.float32)] * NBUF,
        pltpu.VMEM_SHARED((N, DH), jnp.float32),
        [pltpu.SemaphoreType.DMA] * NBUF,
    ]
    if want_deg:
        out_type.append(
            jax.ShapeDtypeStruct((NC, N, 16), jnp.float32))
        scratch_types += [
            pltpu.VMEM((CH, 16), jnp.float32),
            pltpu.VMEM((RQ + TAIL, 16), jnp.float32),
            pltpu.VMEM_SHARED((N, 16), jnp.float32),
        ]
    return pl.kernel(
        functools.partial(_sc_aggregate_body, want_deg=want_deg),
        out_type=out_type,
        mesh=mesh,
        scratch_types=scratch_types,
        compiler_params=pltpu.CompilerParams(use_tc_tiling_on_sc=False),
    )


def _sc_aggregate(xa, xb, src, dst, want_deg):
    return _build_sc_aggregate(want_deg)(xa, xb, src, dst)


def _sc_aggregate_body(*refs, want_deg):
    if want_deg:
        (xa_hbm, xb_hbm, src_hbm, dst_hbm, agga_hbm, aggb_hbm, deg_hbm,
         srcs_v, dsts_v, rows, agg_sh, sems, ones_v, zdeg_v, deg_sh) = refs
    else:
        (xa_hbm, xb_hbm, src_hbm, dst_hbm, agga_hbm, aggb_hbm,
         srcs_v, dsts_v, rows, agg_sh, sems) = refs
    rows_a = rows[0]
    cid = lax.axis_index("c")
    sid = lax.axis_index("s")

    z16 = jnp.zeros((16,), jnp.float32)
    one16 = jnp.full((16,), 1.0, jnp.float32)

    pltpu.sync_copy(src_hbm.at[sid], srcs_v)
    pltpu.sync_copy(dst_hbm.at[sid], dsts_v)

    def _fill_rows(i, _):
        r = i // (DH // 16)
        j = i % (DH // 16)
        rows_a[r, pl.ds(j * 16, 16)] = z16
        return 0
    lax.fori_loop(0, CH * (DH // 16), _fill_rows, 0)

    if want_deg:
        def _fill_ones(r, _):
            ones_v[r, :] = one16
            return 0
        lax.fori_loop(0, CH, _fill_ones, 0)

        def _fill_zdeg(r, _):
            zdeg_v[r, :] = z16
            return 0
        lax.fori_loop(0, RQ + TAIL, _fill_zdeg, 0)

    rbase = sid * RQ
    for k in range(RQ // CH):
        pltpu.sync_copy(rows_a, agg_sh.at[pl.ds(rbase + k * CH, CH)])
    rem = RQ % CH
    if rem:
        pltpu.sync_copy(rows_a.at[pl.ds(0, rem)],
                        agg_sh.at[pl.ds(rbase + (RQ // CH) * CH, rem)])

    @pl.when(sid == NS - 1)
    def _zero_agg_tail():
        pltpu.sync_copy(rows_a.at[pl.ds(0, TAIL)],
                        agg_sh.at[pl.ds(NS * RQ, TAIL)])

    if want_deg:
        pltpu.sync_copy(zdeg_v.at[pl.ds(0, RQ)], deg_sh.at[pl.ds(rbase, RQ)])

        @pl.when(sid == NS - 1)
        def _zero_deg_tail():
            pltpu.sync_copy(zdeg_v.at[pl.ds(0, TAIL)],
                            deg_sh.at[pl.ds(NS * RQ, TAIL)])

    plsc.subcore_barrier()

    def _run_edges(table_hbm):
        def _issue(st, b):
            pltpu.async_copy(table_hbm.at[srcs_v.at[st]], rows[b], sems[b])

        def _wait(st, b):
            pltpu.make_async_copy(
                table_hbm.at[srcs_v.at[st]], rows[b], sems[b]).wait()

        def _scatter(st, b):
            pltpu.sync_copy(rows[b], agg_sh.at[dsts_v.at[st]], add=True)

            if want_deg:
                @pl.when(jnp.logical_xor(cid == 1, st < HSTEP))
                def _deg():
                    pltpu.sync_copy(ones_v, deg_sh.at[dsts_v.at[st]],
                                    add=True)

        for b in range(NBUF):
            _issue(b, b)

        def _outer(q, _):
            for b in range(NBUF):
                st = q * NBUF + b
                _wait(st, b)
                _scatter(st, b)
                _issue(st + NBUF, b)
            return 0
        lax.fori_loop(0, OUTER - 1, _outer, 0)

        for b in range(NBUF):
            st = (OUTER - 1) * NBUF + b
            _wait(st, b)
            _scatter(st, b)

    @pl.when(cid == 0)
    def _edges_a():
        _run_edges(xa_hbm)

    @pl.when(cid == 1)
    def _edges_b():
        _run_edges(xb_hbm)

    plsc.subcore_barrier()

    def _read_out(out_hbm):
        pltpu.sync_copy(agg_sh.at[pl.ds(rbase, RQ)],
                        out_hbm.at[pl.ds(rbase, RQ)])

        @pl.when(sid == NS - 1)
        def _read_tail():
            pltpu.sync_copy(agg_sh.at[pl.ds(NS * RQ, TAIL)],
                            out_hbm.at[pl.ds(NS * RQ, TAIL)])

    @pl.when(cid == 0)
    def _read_a():
        _read_out(agga_hbm)

    @pl.when(cid == 1)
    def _read_b():
        _read_out(aggb_hbm)

    if want_deg:
        pltpu.sync_copy(deg_sh.at[pl.ds(rbase, RQ)],
                        deg_hbm.at[cid, pl.ds(rbase, RQ)])

        @pl.when(sid == NS - 1)
        def _read_deg_tail():
            pltpu.sync_copy(deg_sh.at[pl.ds(NS * RQ, TAIL)],
                            deg_hbm.at[cid, pl.ds(NS * RQ, TAIL)])


def kernel(x, edge_index, W1, b1, W2, b2):
    x = x.astype(jnp.float32)
    src = edge_index[0].astype(jnp.int32).reshape(NS, STEPS, CH)
    dst = edge_index[1].astype(jnp.int32).reshape(NS, STEPS, CH)
    b1r = b1.reshape(1, D).astype(jnp.float32)
    b2r = b2.reshape(1, D).astype(jnp.float32)

    xa1, xb1 = _tc_pre(x, W1, b1r)
    pa1, pb1, deg = _sc_aggregate(xa1, xb1, src, dst, want_deg=True)
    xa2, xb2 = _tc_mid(pa1, pb1, deg, W2, b2r)
    pa2, pb2 = _sc_aggregate(xa2, xb2, src, dst, want_deg=False)
    return _tc_post(pa2, pb2, deg)


# speedup vs baseline: 12.8685x; 1.0394x over previous
import functools
import math

import jax
import jax.numpy as jnp
from jax import lax
from jax.experimental import pallas as pl
from jax.experimental.pallas import tpu as pltpu
from jax.experimental.pallas import tpu_sc as plsc

MIN_NORM = 1e-7
EPS = 4e-3
N = 10000
E = 320000
D = 128
DH = D // 2

NC = 2
NS = 16
EPT = E // NS
CH = 80
STEPS = EPT // CH
NBUF = 5
OUTER = STEPS // NBUF
HSTEP = STEPS // 2
RQ = 624
TAIL = N - NS * RQ

BR = 2000
GRID = N // BR

C_IN, C_HID, C_OUT = 1.0, 1.25, 1.5


def _norm(x):
    return jnp.sqrt(jnp.sum(x * x, axis=-1, keepdims=True))


def _artanh(x):
    x = jnp.clip(x, -1.0 + 1e-7, 1.0 - 1e-7)
    return 0.5 * jnp.log((1.0 + x) / (1.0 - x))


def _proj(x, c):
    norm = jnp.maximum(_norm(x), MIN_NORM)
    maxnorm = (1.0 - EPS) / jnp.sqrt(c)
    return jnp.where(norm > maxnorm, x / norm * maxnorm, x)


def _expmap0(u, c):
    sqrt_c = jnp.sqrt(c)
    u_norm = jnp.maximum(_norm(u), MIN_NORM)
    return jnp.tanh(sqrt_c * u_norm) * u / (sqrt_c * u_norm)


def _logmap0(p, c):
    sqrt_c = jnp.sqrt(c)
    p_norm = jnp.maximum(_norm(p), MIN_NORM)
    return _artanh(sqrt_c * p_norm) * p / (sqrt_c * p_norm)


def _mobius_add(x, y, c):
    x2 = jnp.sum(x * x, axis=-1, keepdims=True)
    y2 = jnp.sum(y * y, axis=-1, keepdims=True)
    xy = jnp.sum(x * y, axis=-1, keepdims=True)
    num = (1.0 + 2.0 * c * xy + c * y2) * x + (1.0 - c * x2) * y
    denom = 1.0 + 2.0 * c * xy + (c ** 2) * x2 * y2
    return num / jnp.maximum(denom, MIN_NORM)


ATANH_LIM = 0.5 * math.log((2.0 - EPS) / EPS)


def _sumb(x, ones_ref):
    return lax.dot_general(x, ones_ref[...], (((1,), (0,)), ((), ())),
                           preferred_element_type=jnp.float32)


def _normb(x, ones_ref):
    return jnp.maximum(jnp.sqrt(_sumb(x * x, ones_ref)), MIN_NORM)


def _exp0_log0b(u, c, ones_ref):
    z = jnp.sqrt(c) * _normb(u, ones_ref)
    return jnp.minimum(z, ATANH_LIM) / z * u


def _expmap0_projb(u, c, ones_ref):
    sqrt_c = jnp.sqrt(c)
    n = _normb(u, ones_ref)
    t = jnp.minimum(jnp.tanh(sqrt_c * n), 1.0 - EPS)
    return t * u / (sqrt_c * n), jnp.maximum(t / sqrt_c, MIN_NORM)


def _hyp_linear_log(h, x_norm, w_ref, b_ref, c, ones_ref):
    sqrt_c = jnp.sqrt(c)
    mx = lax.dot_general(h, w_ref[...], (((1,), (1,)), ((), ())),
                         preferred_element_type=jnp.float32)
    mx_norm = _normb(mx, ones_ref)
    t = jnp.minimum(jnp.tanh(mx_norm / x_norm * _artanh(sqrt_c * x_norm)),
                    1.0 - EPS)
    mv = t * mx / (mx_norm * sqrt_c)
    hyp_bias = _proj(_expmap0(b_ref[...], c), c)
    x2 = t * t / c
    y2 = jnp.sum(hyp_bias * hyp_bias, axis=-1, keepdims=True)
    xy = _sumb(mv * hyp_bias, ones_ref)
    num = (1.0 + 2.0 * c * xy + c * y2) * mv + (1.0 - c * x2) * hyp_bias
    denom = 1.0 + 2.0 * c * xy + (c ** 2) * x2 * y2
    ma = num / jnp.maximum(denom, MIN_NORM)
    zz = sqrt_c * _normb(ma, ones_ref)
    return _artanh(jnp.minimum(zz, 1.0 - EPS)) * ma / zz


def _k_pre(x_ref, w_ref, b_ref, ones_ref, oa_ref, ob_ref):
    h, hn = _expmap0_projb(x_ref[...], C_IN, ones_ref)
    xt = _hyp_linear_log(h, hn, w_ref, b_ref, C_IN, ones_ref)
    oa_ref[...] = xt[:, :DH]
    ob_ref[...] = xt[:, DH:]


def _k_mid(pa_ref, pb_ref, deg_ref, w_ref, b_ref, ones_ref, oa_ref, ob_ref):
    agg = jnp.concatenate([pa_ref[...], pb_ref[...]], axis=-1)
    deg = deg_ref[0, :, 0:1] + deg_ref[1, :, 0:1]
    agg = agg / jnp.maximum(deg, 1.0)
    xt = jax.nn.relu(_exp0_log0b(agg, C_IN, ones_ref))
    h, hn = _expmap0_projb(xt, C_HID, ones_ref)
    xt2 = _hyp_linear_log(h, hn, w_ref, b_ref, C_HID, ones_ref)
    oa_ref[...] = xt2[:, :DH]
    ob_ref[...] = xt2[:, DH:]


def _k_post(pa_ref, pb_ref, deg_ref, ones_ref, o_ref):
    agg = jnp.concatenate([pa_ref[...], pb_ref[...]], axis=-1)
    deg = deg_ref[0, :, 0:1] + deg_ref[1, :, 0:1]
    agg = agg / jnp.maximum(deg, 1.0)
    xt = jax.nn.relu(_exp0_log0b(agg, C_HID, ones_ref))
    o_ref[...] = _expmap0_projb(xt, C_OUT, ones_ref)[0]


def _row_spec():
    return pl.BlockSpec((BR, D), lambda i: (i, 0))


def _half_spec():
    return pl.BlockSpec((BR, DH), lambda i: (i, 0))


def _deg_spec():
    return pl.BlockSpec((NC, BR, 16), lambda i: (0, i, 0))


def _w_spec():
    return pl.BlockSpec((D, D), lambda i: (0, 0))


def _b_spec():
    return pl.BlockSpec((1, D), lambda i: (0, 0))


_half_sds = jax.ShapeDtypeStruct((N, DH), jnp.float32)


def _ones_dd():
    return jnp.ones((D, D), jnp.float32)


def _tc_pre(x, w1, b1):
    return pl.pallas_call(
        _k_pre,
        grid=(GRID,),
        in_specs=[_row_spec(), _w_spec(), _b_spec(), _w_spec()],
        out_specs=[_half_spec(), _half_spec()],
        out_shape=[_half_sds, _half_sds],
    )(x, w1, b1, _ones_dd())


def _tc_mid(pa, pb, deg, w2, b2):
    return pl.pallas_call(
        _k_mid,
        grid=(GRID,),
        in_specs=[_half_spec(), _half_spec(), _deg_spec(), _w_spec(),
                  _b_spec(), _w_spec()],
        out_specs=[_half_spec(), _half_spec()],
        out_shape=[_half_sds, _half_sds],
    )(pa, pb, deg, w2, b2, _ones_dd())


def _tc_post(pa, pb, deg):
    return pl.pallas_call(
        _k_post,
        grid=(GRID,),
        in_specs=[_half_spec(), _half_spec(), _deg_spec(), _w_spec()],
        out_specs=_row_spec(),
        out_shape=jax.ShapeDtypeStruct((N, D), jnp.float32),
    )(pa, pb, deg, _ones_dd())


@functools.cache
def _build_sc_aggregate(want_deg):
    mesh = plsc.VectorSubcoreMesh(
        core_axis_name="c", subcore_axis_name="s",
        num_cores=NC, num_subcores=NS)
    out_type = [
        jax.ShapeDtypeStruct((N, DH), jnp.float32),
        jax.ShapeDtypeStruct((N, DH), jnp.float32),
    ]
    scratch_types = [
        pltpu.VMEM((STEPS, CH), jnp.int32),
        pltpu.VMEM((STEPS, CH), jnp.int32),
        [pltpu.VMEM((CH, DH), jnp.float32)] * NBUF,
        pltpu.VMEM_SHARED((N, DH), jnp.float32),
        [pltpu.SemaphoreType.DMA] * NBUF,
    ]
    if want_deg:
        out_type.append(
            jax.ShapeDtypeStruct((NC, N, 16), jnp.float32))
        scratch_types += [
            pltpu.VMEM((CH, 16), jnp.float32),
            pltpu.VMEM((RQ + TAIL, 16), jnp.float32),
            pltpu.VMEM_SHARED((N, 16), jnp.float32),
        ]
    return pl.kernel(
        functools.partial(_sc_aggregate_body, want_deg=want_deg),
        out_type=out_type,
        mesh=mesh,
        scratch_types=scratch_types,
        compiler_params=pltpu.CompilerParams(use_tc_tiling_on_sc=False),
    )


def _sc_aggregate(xa, xb, src, dst, want_deg):
    return _build_sc_aggregate(want_deg)(xa, xb, src, dst)


def _sc_aggregate_body(*refs, want_deg):
    if want_deg:
        (xa_hbm, xb_hbm, src_hbm, dst_hbm, agga_hbm, aggb_hbm, deg_hbm,
         srcs_v, dsts_v, rows, agg_sh, sems, ones_v, zdeg_v, deg_sh) = refs
    else:
        (xa_hbm, xb_hbm, src_hbm, dst_hbm, agga_hbm, aggb_hbm,
         srcs_v, dsts_v, rows, agg_sh, sems) = refs
    rows_a = rows[0]
    cid = lax.axis_index("c")
    sid = lax.axis_index("s")

    z16 = jnp.zeros((16,), jnp.float32)
    one16 = jnp.full((16,), 1.0, jnp.float32)

    pltpu.sync_copy(src_hbm.at[sid], srcs_v)
    pltpu.sync_copy(dst_hbm.at[sid], dsts_v)

    def _fill_rows(i, _):
        r = i // (DH // 16)
        j = i % (DH // 16)
        rows_a[r, pl.ds(j * 16, 16)] = z16
        return 0
    lax.fori_loop(0, CH * (DH // 16), _fill_rows, 0)

    if want_deg:
        def _fill_ones(r, _):
            ones_v[r, :] = one16
            return 0
        lax.fori_loop(0, CH, _fill_ones, 0)

        def _fill_zdeg(r, _):
            zdeg_v[r, :] = z16
            return 0
        lax.fori_loop(0, RQ + TAIL, _fill_zdeg, 0)

    rbase = sid * RQ
    for k in range(RQ // CH):
        pltpu.sync_copy(rows_a, agg_sh.at[pl.ds(rbase + k * CH, CH)])
    rem = RQ % CH
    if rem:
        pltpu.sync_copy(rows_a.at[pl.ds(0, rem)],
                        agg_sh.at[pl.ds(rbase + (RQ // CH) * CH, rem)])

    @pl.when(sid == NS - 1)
    def _zero_agg_tail():
        pltpu.sync_copy(rows_a.at[pl.ds(0, TAIL)],
                        agg_sh.at[pl.ds(NS * RQ, TAIL)])

    if want_deg:
        pltpu.sync_copy(zdeg_v.at[pl.ds(0, RQ)], deg_sh.at[pl.ds(rbase, RQ)])

        @pl.when(sid == NS - 1)
        def _zero_deg_tail():
            pltpu.sync_copy(zdeg_v.at[pl.ds(0, TAIL)],
                            deg_sh.at[pl.ds(NS * RQ, TAIL)])

    plsc.subcore_barrier()

    def _run_edges(table_hbm):
        def _issue(st, b):
            pltpu.async_copy(table_hbm.at[srcs_v.at[st]], rows[b], sems[b])

        def _wait(st, b):
            pltpu.make_async_copy(
                table_hbm.at[srcs_v.at[st]], rows[b], sems[b]).wait()

        def _scatter(st, b):
            pltpu.sync_copy(rows[b], agg_sh.at[dsts_v.at[st]], add=True)

            if want_deg:
                @pl.when(jnp.logical_xor(cid == 1, st < HSTEP))
                def _deg():
                    pltpu.sync_copy(ones_v, deg_sh.at[dsts_v.at[st]],
                                    add=True)

        for b in range(NBUF):
            _issue(b, b)

        def _outer(q, _):
            for b in range(NBUF):
                st = q * NBUF + b
                _wait(st, b)
                _scatter(st, b)
                _issue(st + NBUF, b)
            return 0
        lax.fori_loop(0, OUTER - 1, _outer, 0)

        for b in range(NBUF):
            st = (OUTER - 1) * NBUF + b
            _wait(st, b)
            _scatter(st, b)

    @pl.when(cid == 0)
    def _edges_a():
        _run_edges(xa_hbm)

    @pl.when(cid == 1)
    def _edges_b():
        _run_edges(xb_hbm)

    plsc.subcore_barrier()

    def _read_out(out_hbm):
        pltpu.sync_copy(agg_sh.at[pl.ds(rbase, RQ)],
                        out_hbm.at[pl.ds(rbase, RQ)])

        @pl.when(sid == NS - 1)
        def _read_tail():
            pltpu.sync_copy(agg_sh.at[pl.ds(NS * RQ, TAIL)],
                            out_hbm.at[pl.ds(NS * RQ, TAIL)])

    @pl.when(cid == 0)
    def _read_a():
        _read_out(agga_hbm)

    @pl.when(cid == 1)
    def _read_b():
        _read_out(aggb_hbm)

    if want_deg:
        pltpu.sync_copy(deg_sh.at[pl.ds(rbase, RQ)],
                        deg_hbm.at[cid, pl.ds(rbase, RQ)])

        @pl.when(sid == NS - 1)
        def _read_deg_tail():
            pltpu.sync_copy(deg_sh.at[pl.ds(NS * RQ, TAIL)],
                            deg_hbm.at[cid, pl.ds(NS * RQ, TAIL)])


def kernel(x, edge_index, W1, b1, W2, b2):
    x = x.astype(jnp.float32)
    src = edge_index[0].astype(jnp.int32).reshape(NS, STEPS, CH)
    dst = edge_index[1].astype(jnp.int32).reshape(NS, STEPS, CH)
    b1r = b1.reshape(1, D).astype(jnp.float32)
    b2r = b2.reshape(1, D).astype(jnp.float32)

    xa1, xb1 = _tc_pre(x, W1, b1r)
    pa1, pb1, deg = _sc_aggregate(xa1, xb1, src, dst, want_deg=True)
    xa2, xb2 = _tc_mid(pa1, pb1, deg, W2, b2r)
    pa2, pb2 = _sc_aggregate(xa2, xb2, src, dst, want_deg=False)
    return _tc_post(pa2, pb2, deg)


# speedup vs baseline: 13.1491x; 1.0218x over previous
import functools
import math

import jax
import jax.numpy as jnp
from jax import lax
from jax.experimental import pallas as pl
from jax.experimental.pallas import tpu as pltpu
from jax.experimental.pallas import tpu_sc as plsc

MIN_NORM = 1e-7
EPS = 4e-3
N = 10000
E = 320000
D = 128
DH = D // 2

NC = 2
NS = 16
EPT = E // NS
CH = 80
STEPS = EPT // CH
NBUF = 5
OUTER = STEPS // NBUF
HSTEP = STEPS // 2
RQ = 624
TAIL = N - NS * RQ

BR = 2000
GRID = N // BR

C_IN, C_HID, C_OUT = 1.0, 1.25, 1.5


def _norm(x):
    return jnp.sqrt(jnp.sum(x * x, axis=-1, keepdims=True))


def _artanh(x):
    x = jnp.clip(x, -1.0 + 1e-7, 1.0 - 1e-7)
    return 0.5 * jnp.log((1.0 + x) / (1.0 - x))


def _proj(x, c):
    norm = jnp.maximum(_norm(x), MIN_NORM)
    maxnorm = (1.0 - EPS) / jnp.sqrt(c)
    return jnp.where(norm > maxnorm, x / norm * maxnorm, x)


def _expmap0(u, c):
    sqrt_c = jnp.sqrt(c)
    u_norm = jnp.maximum(_norm(u), MIN_NORM)
    return jnp.tanh(sqrt_c * u_norm) * u / (sqrt_c * u_norm)


def _logmap0(p, c):
    sqrt_c = jnp.sqrt(c)
    p_norm = jnp.maximum(_norm(p), MIN_NORM)
    return _artanh(sqrt_c * p_norm) * p / (sqrt_c * p_norm)


def _mobius_add(x, y, c):
    x2 = jnp.sum(x * x, axis=-1, keepdims=True)
    y2 = jnp.sum(y * y, axis=-1, keepdims=True)
    xy = jnp.sum(x * y, axis=-1, keepdims=True)
    num = (1.0 + 2.0 * c * xy + c * y2) * x + (1.0 - c * x2) * y
    denom = 1.0 + 2.0 * c * xy + (c ** 2) * x2 * y2
    return num / jnp.maximum(denom, MIN_NORM)


ATANH_LIM = 0.5 * math.log((2.0 - EPS) / EPS)


def _sumb(x, ones_ref):
    return lax.dot_general(x, ones_ref[...], (((1,), (0,)), ((), ())),
                           preferred_element_type=jnp.float32)


def _normb(x, ones_ref):
    return jnp.maximum(jnp.sqrt(_sumb(x * x, ones_ref)), MIN_NORM)


def _exp0_log0b(u, c, ones_ref):
    z = jnp.sqrt(c) * _normb(u, ones_ref)
    return jnp.minimum(z, ATANH_LIM) / z * u


def _expmap0_projb(u, c, ones_ref):
    sqrt_c = jnp.sqrt(c)
    n = _normb(u, ones_ref)
    z = sqrt_c * n
    t = jnp.minimum(jnp.tanh(z), 1.0 - EPS)
    return (t * u / z,
            jnp.maximum(t / sqrt_c, MIN_NORM),
            jnp.minimum(z, ATANH_LIM))


def _hyp_linear_log(h, x_norm, x_atanh, w_ref, b_ref, c, ones_ref):
    sqrt_c = jnp.sqrt(c)
    mx = lax.dot_general(h, w_ref[...], (((1,), (1,)), ((), ())),
                         preferred_element_type=jnp.float32)
    mx_norm = _normb(mx, ones_ref)
    t = jnp.minimum(jnp.tanh(mx_norm / x_norm * x_atanh), 1.0 - EPS)
    mv = t * mx / (mx_norm * sqrt_c)
    hyp_bias = _proj(_expmap0(b_ref[...], c), c)
    x2 = t * t / c
    y2 = jnp.sum(hyp_bias * hyp_bias, axis=-1, keepdims=True)
    xy = _sumb(mv * hyp_bias, ones_ref)
    num = (1.0 + 2.0 * c * xy + c * y2) * mv + (1.0 - c * x2) * hyp_bias
    denom = 1.0 + 2.0 * c * xy + (c ** 2) * x2 * y2
    ma = num / jnp.maximum(denom, MIN_NORM)
    zz = sqrt_c * _normb(ma, ones_ref)
    return _artanh(jnp.minimum(zz, 1.0 - EPS)) * ma / zz


def _k_pre(x_ref, w_ref, b_ref, ones_ref, oa_ref, ob_ref):
    h, hn, ha = _expmap0_projb(x_ref[...], C_IN, ones_ref)
    xt = _hyp_linear_log(h, hn, ha, w_ref, b_ref, C_IN, ones_ref)
    oa_ref[...] = xt[:, :DH]
    ob_ref[...] = xt[:, DH:]


def _k_mid(pa_ref, pb_ref, deg_ref, w_ref, b_ref, ones_ref, oa_ref, ob_ref):
    agg = jnp.concatenate([pa_ref[...], pb_ref[...]], axis=-1)
    deg = deg_ref[0, :, 0:1] + deg_ref[1, :, 0:1]
    agg = agg / jnp.maximum(deg, 1.0)
    xt = jax.nn.relu(_exp0_log0b(agg, C_IN, ones_ref))
    h, hn, ha = _expmap0_projb(xt, C_HID, ones_ref)
    xt2 = _hyp_linear_log(h, hn, ha, w_ref, b_ref, C_HID, ones_ref)
    oa_ref[...] = xt2[:, :DH]
    ob_ref[...] = xt2[:, DH:]


def _k_post(pa_ref, pb_ref, deg_ref, ones_ref, o_ref):
    agg = jnp.concatenate([pa_ref[...], pb_ref[...]], axis=-1)
    deg = deg_ref[0, :, 0:1] + deg_ref[1, :, 0:1]
    agg = agg / jnp.maximum(deg, 1.0)
    xt = jax.nn.relu(_exp0_log0b(agg, C_HID, ones_ref))
    o_ref[...] = _expmap0_projb(xt, C_OUT, ones_ref)[0]


def _row_spec():
    return pl.BlockSpec((BR, D), lambda i: (i, 0))


def _half_spec():
    return pl.BlockSpec((BR, DH), lambda i: (i, 0))


def _deg_spec():
    return pl.BlockSpec((NC, BR, 16), lambda i: (0, i, 0))


def _w_spec():
    return pl.BlockSpec((D, D), lambda i: (0, 0))


def _b_spec():
    return pl.BlockSpec((1, D), lambda i: (0, 0))


_half_sds = jax.ShapeDtypeStruct((N, DH), jnp.float32)


def _ones_dd():
    return jnp.ones((D, D), jnp.float32)


def _tc_pre(x, w1, b1):
    return pl.pallas_call(
        _k_pre,
        grid=(GRID,),
        in_specs=[_row_spec(), _w_spec(), _b_spec(), _w_spec()],
        out_specs=[_half_spec(), _half_spec()],
        out_shape=[_half_sds, _half_sds],
    )(x, w1, b1, _ones_dd())


def _tc_mid(pa, pb, deg, w2, b2):
    return pl.pallas_call(
        _k_mid,
        grid=(GRID,),
        in_specs=[_half_spec(), _half_spec(), _deg_spec(), _w_spec(),
                  _b_spec(), _w_spec()],
        out_specs=[_half_spec(), _half_spec()],
        out_shape=[_half_sds, _half_sds],
    )(pa, pb, deg, w2, b2, _ones_dd())


def _tc_post(pa, pb, deg):
    return pl.pallas_call(
        _k_post,
        grid=(GRID,),
        in_specs=[_half_spec(), _half_spec(), _deg_spec(), _w_spec()],
        out_specs=_row_spec(),
        out_shape=jax.ShapeDtypeStruct((N, D), jnp.float32),
    )(pa, pb, deg, _ones_dd())


@functools.cache
def _build_sc_aggregate(want_deg):
    mesh = plsc.VectorSubcoreMesh(
        core_axis_name="c", subcore_axis_name="s",
        num_cores=NC, num_subcores=NS)
    out_type = [
        jax.ShapeDtypeStruct((N, DH), jnp.float32),
        jax.ShapeDtypeStruct((N, DH), jnp.float32),
    ]
    scratch_types = [
        pltpu.VMEM((STEPS, CH), jnp.int32),
        pltpu.VMEM((STEPS, CH), jnp.int32),
        [pltpu.VMEM((CH, DH), jnp.float32)] * NBUF,
        pltpu.VMEM_SHARED((N, DH), jnp.float32),
        [pltpu.SemaphoreType.DMA] * NBUF,
    ]
    if want_deg:
        out_type.append(
            jax.ShapeDtypeStruct((NC, N, 16), jnp.float32))
        scratch_types += [
            pltpu.VMEM((CH, 16), jnp.float32),
            pltpu.VMEM((CH, 16), jnp.float32),
            pltpu.VMEM_SHARED((N, 16), jnp.float32),
        ]
    return pl.kernel(
        functools.partial(_sc_aggregate_body, want_deg=want_deg),
        out_type=out_type,
        mesh=mesh,
        scratch_types=scratch_types,
        compiler_params=pltpu.CompilerParams(use_tc_tiling_on_sc=False),
    )


def _sc_aggregate(xa, xb, src, dst, want_deg):
    return _build_sc_aggregate(want_deg)(xa, xb, src, dst)


def _sc_aggregate_body(*refs, want_deg):
    if want_deg:
        (xa_hbm, xb_hbm, src_hbm, dst_hbm, agga_hbm, aggb_hbm, deg_hbm,
         srcs_v, dsts_v, rows, agg_sh, sems, ones_v, zdeg_v, deg_sh) = refs
    else:
        (xa_hbm, xb_hbm, src_hbm, dst_hbm, agga_hbm, aggb_hbm,
         srcs_v, dsts_v, rows, agg_sh, sems) = refs
    rows_a = rows[0]
    cid = lax.axis_index("c")
    sid = lax.axis_index("s")

    z16 = jnp.zeros((16,), jnp.float32)
    one16 = jnp.full((16,), 1.0, jnp.float32)

    def _issue(table, st, b):
        pltpu.async_copy(table.at[srcs_v.at[st]], rows[b], sems[b])

    def _wait(table, st, b):
        pltpu.make_async_copy(
            table.at[srcs_v.at[st]], rows[b], sems[b]).wait()

    pltpu.sync_copy(src_hbm.at[sid], srcs_v)

    @pl.when(cid == 0)
    def _prime_a():
        for b in range(1, NBUF):
            _issue(xa_hbm, b, b)

    @pl.when(cid == 1)
    def _prime_b():
        for b in range(1, NBUF):
            _issue(xb_hbm, b, b)

    def _fill_rows(i, _):
        r = i // (DH // 16)
        j = i % (DH // 16)
        rows_a[r, pl.ds(j * 16, 16)] = z16
        return 0
    lax.fori_loop(0, CH * (DH // 16), _fill_rows, 0)

    if want_deg:
        def _fill_ones(r, _):
            ones_v[r, :] = one16
            zdeg_v[r, :] = z16
            return 0
        lax.fori_loop(0, CH, _fill_ones, 0)

    pltpu.sync_copy(dst_hbm.at[sid], dsts_v)

    rbase = sid * RQ
    for k in range(RQ // CH):
        pltpu.sync_copy(rows_a, agg_sh.at[pl.ds(rbase + k * CH, CH)])
    rem = RQ % CH
    if rem:
        pltpu.sync_copy(rows_a.at[pl.ds(0, rem)],
                        agg_sh.at[pl.ds(rbase + (RQ // CH) * CH, rem)])

    @pl.when(sid == NS - 1)
    def _zero_agg_tail():
        pltpu.sync_copy(rows_a.at[pl.ds(0, TAIL)],
                        agg_sh.at[pl.ds(NS * RQ, TAIL)])

    if want_deg:
        for k in range(RQ // CH):
            pltpu.sync_copy(zdeg_v, deg_sh.at[pl.ds(rbase + k * CH, CH)])
        if rem:
            pltpu.sync_copy(zdeg_v.at[pl.ds(0, rem)],
                            deg_sh.at[pl.ds(rbase + (RQ // CH) * CH, rem)])

        @pl.when(sid == NS - 1)
        def _zero_deg_tail():
            pltpu.sync_copy(zdeg_v.at[pl.ds(0, TAIL)],
                            deg_sh.at[pl.ds(NS * RQ, TAIL)])

    @pl.when(cid == 0)
    def _prime_a0():
        _issue(xa_hbm, 0, 0)

    @pl.when(cid == 1)
    def _prime_b0():
        _issue(xb_hbm, 0, 0)

    plsc.subcore_barrier()

    def _run_edges(table):
        def _scatter(st, b):
            pltpu.sync_copy(rows[b], agg_sh.at[dsts_v.at[st]], add=True)

            if want_deg:
                @pl.when(jnp.logical_xor(cid == 1, st < HSTEP))
                def _deg():
                    pltpu.sync_copy(ones_v, deg_sh.at[dsts_v.at[st]],
                                    add=True)

        def _outer(q, _):
            for b in range(NBUF):
                st = q * NBUF + b
                _wait(table, st, b)
                _scatter(st, b)
                _issue(table, st + NBUF, b)
            return 0
        lax.fori_loop(0, OUTER - 1, _outer, 0)

        for b in range(NBUF):
            st = (OUTER - 1) * NBUF + b
            _wait(table, st, b)
            _scatter(st, b)

    @pl.when(cid == 0)
    def _edges_a():
        _run_edges(xa_hbm)

    @pl.when(cid == 1)
    def _edges_b():
        _run_edges(xb_hbm)

    plsc.subcore_barrier()

    def _read_out(out_hbm):
        pltpu.sync_copy(agg_sh.at[pl.ds(rbase, RQ)],
                        out_hbm.at[pl.ds(rbase, RQ)])

        @pl.when(sid == NS - 1)
        def _read_tail():
            pltpu.sync_copy(agg_sh.at[pl.ds(NS * RQ, TAIL)],
                            out_hbm.at[pl.ds(NS * RQ, TAIL)])

    @pl.when(cid == 0)
    def _read_a():
        _read_out(agga_hbm)

    @pl.when(cid == 1)
    def _read_b():
        _read_out(aggb_hbm)

    if want_deg:
        pltpu.sync_copy(deg_sh.at[pl.ds(rbase, RQ)],
                        deg_hbm.at[cid, pl.ds(rbase, RQ)])

        @pl.when(sid == NS - 1)
        def _read_deg_tail():
            pltpu.sync_copy(deg_sh.at[pl.ds(NS * RQ, TAIL)],
                            deg_hbm.at[cid, pl.ds(NS * RQ, TAIL)])


def kernel(x, edge_index, W1, b1, W2, b2):
    x = x.astype(jnp.float32)
    src = edge_index[0].astype(jnp.int32).reshape(NS, STEPS, CH)
    dst = edge_index[1].astype(jnp.int32).reshape(NS, STEPS, CH)
    b1r = b1.reshape(1, D).astype(jnp.float32)
    b2r = b2.reshape(1, D).astype(jnp.float32)

    xa1, xb1 = _tc_pre(x, W1, b1r)
    pa1, pb1, deg = _sc_aggregate(xa1, xb1, src, dst, want_deg=True)
    xa2, xb2 = _tc_mid(pa1, pb1, deg, W2, b2r)
    pa2, pb2 = _sc_aggregate(xa2, xb2, src, dst, want_deg=False)
    return _tc_post(pa2, pb2, deg)
